# baseline (speedup 1.0000x reference)
import functools

import jax
import jax.numpy as jnp
from jax import lax
from jax.experimental import pallas as pl
from jax.experimental.pallas import tpu as pltpu
from jax.experimental.pallas import tpu_sc as plsc

NUM_CLASSES = 1000
INPUT_SIZE = 200
BATCH = 16384

_CT = 128
_OVL = 2 * _CT - INPUT_SIZE

_NC = 2
_NS = 16
_NW = _NC * _NS
_B_PER_W = BATCH // _NW
_CHUNK = 128
_NCHUNK = _B_PER_W // _CHUNK

_GRID = 32
_BLK_B = BATCH // _GRID


def _split_body(w_ref, a_ref, b_ref):
    x = w_ref[...]
    s = 1.0 / (1.0 + jnp.exp(-x))
    a_ref[...] = s[:, :_CT]
    b_ref[...] = s[:, INPUT_SIZE - _CT:]


_splitter = pl.pallas_call(
    _split_body,
    out_shape=(
        jax.ShapeDtypeStruct((NUM_CLASSES, _CT), jnp.float32),
        jax.ShapeDtypeStruct((NUM_CLASSES, _CT), jnp.float32),
    ),
)


def _gather_body(idx_hbm, ta_hbm, tb_hbm, oa_hbm, ob_hbm,
                 idx_v, a0, a1, b0, b1, sem_a0, sem_a1, sem_b0, sem_b1):
    wid = lax.axis_index("s") * _NC + lax.axis_index("c")
    base = wid * _B_PER_W
    pltpu.sync_copy(idx_hbm.at[pl.ds(base, _B_PER_W)], idx_v)
    abufs, bbufs = (a0, a1), (b0, b1)
    asems, bsems = (sem_a0, sem_a1), (sem_b0, sem_b1)
    acp = [None, None]
    bcp = [None, None]
    idx0 = idx_v.at[pl.ds(0, _CHUNK)]
    acp[0] = pltpu.async_copy(ta_hbm.at[idx0], abufs[0], asems[0])
    bcp[0] = pltpu.async_copy(tb_hbm.at[idx0], bbufs[0], bsems[0])
    for k in range(1, _NCHUNK + 1):
        if k < _NCHUNK:
            idx_k = idx_v.at[pl.ds(k * _CHUNK, _CHUNK)]
            j = k % 2
            acp[j] = pltpu.async_copy(ta_hbm.at[idx_k], abufs[j], asems[j])
            bcp[j] = pltpu.async_copy(tb_hbm.at[idx_k], bbufs[j], bsems[j])
        j = (k - 1) % 2
        rows = pl.ds(base + (k - 1) * _CHUNK, _CHUNK)
        acp[j].wait()
        pltpu.sync_copy(abufs[j], oa_hbm.at[rows])
        bcp[j].wait()
        pltpu.sync_copy(bbufs[j], ob_hbm.at[rows])


@functools.cache
def _gather_halves():
    return pl.kernel(
        _gather_body,
        out_type=(
            jax.ShapeDtypeStruct((BATCH, _CT), jnp.float32),
            jax.ShapeDtypeStruct((BATCH, _CT), jnp.float32),
        ),
        mesh=plsc.VectorSubcoreMesh(core_axis_name="c", subcore_axis_name="s"),
        scratch_types=[
            pltpu.VMEM((_B_PER_W,), jnp.int32),
            pltpu.VMEM((_CHUNK, _CT), jnp.float32),
            pltpu.VMEM((_CHUNK, _CT), jnp.float32),
            pltpu.VMEM((_CHUNK, _CT), jnp.float32),
            pltpu.VMEM((_CHUNK, _CT), jnp.float32),
            pltpu.SemaphoreType.DMA,
            pltpu.SemaphoreType.DMA,
            pltpu.SemaphoreType.DMA,
            pltpu.SemaphoreType.DMA,
        ],
        compiler_params=pltpu.CompilerParams(use_tc_tiling_on_sc=False),
    )


def _finish_body(a_ref, b_ref, o_ref):
    o_ref[...] = jnp.concatenate(
        [a_ref[...], b_ref[...][:, _OVL:]], axis=1)


_finisher = pl.pallas_call(
    _finish_body,
    grid=(_GRID,),
    in_specs=[
        pl.BlockSpec((_BLK_B, _CT), lambda g: (g, 0)),
        pl.BlockSpec((_BLK_B, _CT), lambda g: (g, 0)),
    ],
    out_specs=pl.BlockSpec((_BLK_B, INPUT_SIZE), lambda g: (g, 0)),
    out_shape=jax.ShapeDtypeStruct((BATCH, INPUT_SIZE), jnp.float32),
)


def kernel(class_labels, class_weights):
    ta, tb = _splitter(class_weights)
    oa, ob = _gather_halves()(class_labels.astype(jnp.int32), ta, tb)
    return _finisher(oa, ob)


# speedup vs baseline: 1.1105x; 1.1105x over previous
import functools

import jax
import jax.numpy as jnp
from jax import lax
from jax.experimental import pallas as pl
from jax.experimental.pallas import tpu as pltpu
from jax.experimental.pallas import tpu_sc as plsc

NUM_CLASSES = 1000
INPUT_SIZE = 200
BATCH = 16384

_VPAD = 1024
_NC = 2
_NS = 16
_NW = _NC * _NS

_RG = 8
_BG = BATCH // 2
_NUNITS = (INPUT_SIZE // _RG) * (BATCH // _BG)
_NJ = _BG // 16


def _prep_body(wt_ref, o_ref):
    o_ref[:, :NUM_CLASSES] = 1.0 / (1.0 + jnp.exp(-wt_ref[...]))
    o_ref[:, NUM_CLASSES:] = jnp.zeros(
        (INPUT_SIZE, _VPAD - NUM_CLASSES), jnp.float32)


_prep = pl.pallas_call(
    _prep_body,
    out_shape=jax.ShapeDtypeStruct((INPUT_SIZE, _VPAD), jnp.float32),
)


def _unit(u, idx_hbm, tt_hbm, out_hbm, tv, iv, ov):
    g = u // 2
    b0 = (u % 2) * _BG
    pltpu.sync_copy(tt_hbm.at[pl.ds(g * _RG, _RG)], tv)
    pltpu.sync_copy(idx_hbm.at[pl.ds(b0, _BG)], iv)

    def j_body(j, carry):
        idx16 = iv[pl.ds(j * 16, 16)]
        for r in range(_RG):
            row = jnp.full((16,), r, jnp.int32)
            ov[r, pl.ds(j * 16, 16)] = plsc.load_gather(tv, [row, idx16])
        return carry

    lax.fori_loop(0, _NJ, j_body, 0)
    pltpu.sync_copy(ov, out_hbm.at[pl.ds(g * _RG, _RG), pl.ds(b0, _BG)])


def _gather_body(idx_hbm, tt_hbm, out_hbm, tv, iv, ov):
    wid = lax.axis_index("s") * _NC + lax.axis_index("c")
    _unit(wid, idx_hbm, tt_hbm, out_hbm, tv, iv, ov)

    @pl.when(wid + _NW < _NUNITS)
    def _():
        _unit(wid + _NW, idx_hbm, tt_hbm, out_hbm, tv, iv, ov)


@functools.cache
def _gather_t():
    return pl.kernel(
        _gather_body,
        out_type=jax.ShapeDtypeStruct((INPUT_SIZE, BATCH), jnp.float32),
        mesh=plsc.VectorSubcoreMesh(core_axis_name="c", subcore_axis_name="s"),
        scratch_types=[
            pltpu.VMEM((_RG, _VPAD), jnp.float32),
            pltpu.VMEM((_BG,), jnp.int32),
            pltpu.VMEM((_RG, _BG), jnp.float32),
        ],
        compiler_params=pltpu.CompilerParams(
            use_tc_tiling_on_sc=True, needs_layout_passes=False),
    )


def kernel(class_labels, class_weights):
    tt = _prep(jnp.transpose(class_weights))
    out_t = _gather_t()(class_labels.astype(jnp.int32), tt)
    return jnp.transpose(out_t)


# speedup vs baseline: 1.8422x; 1.6588x over previous
import functools

import jax
import jax.numpy as jnp
from jax import lax
from jax.experimental import pallas as pl
from jax.experimental.pallas import tpu as pltpu
from jax.experimental.pallas import tpu_sc as plsc

NUM_CLASSES = 1000
INPUT_SIZE = 200
BATCH = 16384

_VPAD = 1024
_NC = 2
_NS = 16
_NW = _NC * _NS

_RG = 8
_BG = 2048
_NCH = BATCH // _BG
_NUNITS = (INPUT_SIZE // _RG) * _NCH
_NJ = _BG // 16


def _prep_body(wt_ref, o_ref):
    o_ref[:, :NUM_CLASSES] = 1.0 / (1.0 + jnp.exp(-wt_ref[...]))
    o_ref[:, NUM_CLASSES:] = jnp.zeros(
        (INPUT_SIZE, _VPAD - NUM_CLASSES), jnp.float32)


_prep = pl.pallas_call(
    _prep_body,
    out_shape=jax.ShapeDtypeStruct((INPUT_SIZE, _VPAD), jnp.float32),
)


def _gather_body(idx_hbm, tt_hbm, out_hbm, tv, iv, ov):
    wid = lax.axis_index("s") * _NC + lax.axis_index("c")
    lo = wid * _NUNITS // _NW
    hi = (wid + 1) * _NUNITS // _NW
    rows = [jnp.full((16,), r, jnp.int32) for r in range(_RG)]

    def unit_body(u, carry):
        g = u // _NCH
        ch = u % _NCH
        pltpu.sync_copy(tt_hbm.at[pl.ds(g * _RG, _RG)], tv)
        pltpu.sync_copy(idx_hbm.at[pl.ds(ch * _BG, _BG)], iv)

        @plsc.parallel_loop(0, _NJ, unroll=4)
        def _(j):
            idx16 = iv[pl.ds(j * 16, 16)]
            for r in range(_RG):
                ov[r, pl.ds(j * 16, 16)] = plsc.load_gather(
                    tv, [rows[r], idx16])

        pltpu.sync_copy(ov, out_hbm.at[pl.ds(g * _RG, _RG),
                                       pl.ds(ch * _BG, _BG)])
        return carry

    lax.fori_loop(lo, hi, unit_body, 0)


@functools.cache
def _gather_t():
    return pl.kernel(
        _gather_body,
        out_type=jax.ShapeDtypeStruct((INPUT_SIZE, BATCH), jnp.float32),
        mesh=plsc.VectorSubcoreMesh(core_axis_name="c", subcore_axis_name="s"),
        scratch_types=[
            pltpu.VMEM((_RG, _VPAD), jnp.float32),
            pltpu.VMEM((_BG,), jnp.int32),
            pltpu.VMEM((_RG, _BG), jnp.float32),
        ],

        compiler_params=pltpu.CompilerParams(
            use_tc_tiling_on_sc=True, needs_layout_passes=False),
    )


def kernel(class_labels, class_weights):
    tt = _prep(jnp.transpose(class_weights))
    out_t = _gather_t()(class_labels.astype(jnp.int32), tt)
    return jnp.transpose(out_t)
